# Initial kernel scaffold; baseline (speedup 1.0000x reference)
import jax
import jax.numpy as jnp
from jax.experimental import pallas as pl


def kernel(h, phrase_mask, phrase_token_idx, phrase_end_pos, rope_cos, rope_sin, W_dq, W_uq, kv_Wkv, kv_Wz, kv_Bpos, ik_Wkv, ik_Wz, ik_Bpos, idx_Wiuq, idx_Ww, qn_w, kn_w, W_o, sink_logits, norm_w, sw_Wq, sw_Wk, sw_Wv, sw_Wo):
    raise NotImplementedError("write your pallas kernel here")


# speedup vs baseline: 6.0670x; 6.0670x over previous
import functools
import math

import jax
import jax.numpy as jnp
from jax.experimental import pallas as pl

T = 2048
D = 1024
H, C = 16, 64
QCD = 256
TOPK = 32
NI, DI = 4, 64
LMAX = 16
P = 512
NWIN = 128
HS = 8

TB = 256
NTB = T // TB
PB = 32
NPB = P // PB
NEG = -1e30

f32 = jnp.float32
bf16 = jnp.bfloat16
i32 = jnp.int32


def _dot(a, b):
    return jax.lax.dot_general(a, b, (((1,), (0,)), ((), ())),
                               preferred_element_type=f32)


def _dot_nt(a, b):
    return jax.lax.dot_general(a, b, (((1,), (1,)), ((), ())),
                               preferred_element_type=f32)


def _rope(x, cos, sin):
    half = x.shape[-1] // 2
    rot = jnp.concatenate([-x[:, half:], x[:, :half]], axis=1)
    return x * cos + rot * sin


def _proj_body(h_ref, nw_ref, wzc_ref, wdq_ref, wiuq_ref, wuq_ref, www_ref,
               wsq_ref, wsk_ref, wsv_ref,
               zc4_ref, qi_ref, wh_ref, qraw_ref, xqs_ref, xks_ref, xvs_ref):
    hb = h_ref[...]
    ms = jnp.mean(hb * hb, axis=1, keepdims=True)
    x = hb * jax.lax.rsqrt(ms + 1e-6) * nw_ref[...]
    xb = x.astype(bf16)
    zc4_ref[...] = _dot(xb, wzc_ref[...])
    qlat = _dot(xb, wdq_ref[...])
    qlb = qlat.astype(bf16)
    qi_ref[...] = _dot(qlb, wiuq_ref[...]).astype(bf16)
    qraw_ref[...] = _dot(qlb, wuq_ref[...])
    wh_ref[...] = _dot(xb, www_ref[...])
    xqs_ref[...] = _dot(xb, wsq_ref[...]).astype(bf16)
    xks_ref[...] = _dot(xb, wsk_ref[...]).astype(bf16)
    xvs_ref[...] = _dot(xb, wsv_ref[...]).astype(bf16)


def _compress_body(idx_ref, zc4_ref, endp_ref, bkv_ref, bik_ref, cs_ref,
                   knw_ref, ccomp_ref, kidx_ref, kall_ref):
    idxv = idx_ref[...]
    ioh = jax.lax.broadcasted_iota(i32, (PB * LMAX, T), 1)
    oh = (idxv == ioh).astype(bf16)
    g = _dot(oh, zc4_ref[...].astype(bf16))

    def slot(l, lo):
        return g[l * PB:(l + 1) * PB, lo:lo + 64]

    mkv = jnp.full((PB, 64), NEG, f32)
    mik = jnp.full((PB, 64), NEG, f32)
    for l in range(LMAX):
        mkv = jnp.maximum(mkv, slot(l, 64) + bkv_ref[l:l + 1, :])
        mik = jnp.maximum(mik, slot(l, 192) + bik_ref[l:l + 1, :])
    skv = jnp.zeros((PB, 64), f32)
    sik = jnp.zeros((PB, 64), f32)
    akv = jnp.zeros((PB, 64), f32)
    aik = jnp.zeros((PB, 64), f32)
    for l in range(LMAX):
        ekv = jnp.exp(slot(l, 64) + bkv_ref[l:l + 1, :] - mkv)
        eik = jnp.exp(slot(l, 192) + bik_ref[l:l + 1, :] - mik)
        skv += ekv
        sik += eik
        akv += ekv * slot(l, 0)
        aik += eik * slot(l, 128)
    ccomp = akv / skv
    ccomp_ref[...] = ccomp
    kidx_ref[...] = aik / sik

    ohe = (endp_ref[...] == jax.lax.broadcasted_iota(i32, (PB, T), 1)).astype(bf16)
    cse = _dot(ohe, cs_ref[...])
    mean = jnp.mean(ccomp * ccomp, axis=1, keepdims=True)
    kn = ccomp * jax.lax.rsqrt(mean + 1e-6) * knw_ref[...]
    kall_ref[...] = _rope(kn, cse[:, :64], cse[:, 64:]) * (1.0 / math.sqrt(C))


def _topk_body(qi_ref, wh_ref, kidx_ref, endp_ref, mask_ref):
    kb = kidx_ref[...].astype(bf16)
    scores = jnp.zeros((TB, P), f32)
    for ih in range(NI):
        s = _dot_nt(qi_ref[:, 64 * ih:64 * ih + 64], kb)
        scores += jnp.maximum(s, 0.0) * wh_ref[:, ih:ih + 1]
    t0 = pl.program_id(0) * TB
    rowpos = (t0 + jax.lax.broadcasted_iota(i32, (TB, 1), 0)).astype(f32)
    vis = endp_ref[...] < rowpos
    scores = jnp.where(vis, scores, NEG)
    lane = jax.lax.broadcasted_iota(i32, (TB, P), 1)
    selb = jnp.zeros((TB, P), jnp.bool_)
    for _ in range(TOPK):
        m = jnp.max(scores, axis=1, keepdims=True)
        valid = m > -1e29
        ism = scores == m
        ii = jnp.min(jnp.where(ism, lane, P * 8), axis=1, keepdims=True)
        s1 = (lane == ii) & valid
        selb = selb | s1
        scores = jnp.where(s1, -3e30, scores)
    mask_ref[...] = jnp.where(selb, 0.0, NEG)


def _swin_body(xqs_ref, xks_ref, xksp_ref, xvs_ref, xvsp_ref, cs_ref, csp_ref,
               wo_ref, osw_ref):
    blk = pl.program_id(0)
    csc = cs_ref[...]
    cosc, sinc = csc[:, :64], csc[:, 64:]
    csp = csp_ref[...]
    cosp, sinp = csp[:, :64], csp[:, 64:]
    isub = jax.lax.broadcasted_iota(i32, (TB, TB), 0)
    jlan = jax.lax.broadcasted_iota(i32, (TB, TB), 1)
    mask_cur = (jlan <= isub) & (isub - jlan < NWIN)
    mask_prev = (isub < jlan - (TB - NWIN)) & (blk > 0)
    outs = []
    for hh in range(HS):
        sl = slice(64 * hh, 64 * hh + 64)
        q = _rope(xqs_ref[:, sl].astype(f32), cosc, sinc) * (1.0 / math.sqrt(C))
        kc = _rope(xks_ref[:, sl].astype(f32), cosc, sinc)
        kp = _rope(xksp_ref[:, sl].astype(f32), cosp, sinp)
        lc = _dot_nt(q.astype(bf16), kc.astype(bf16))
        lp = _dot_nt(q.astype(bf16), kp.astype(bf16))
        lc = jnp.where(mask_cur, lc, NEG)
        lp = jnp.where(mask_prev, lp, NEG)
        m = jnp.maximum(jnp.max(lc, axis=1, keepdims=True),
                        jnp.max(lp, axis=1, keepdims=True))
        ec = jnp.exp(lc - m)
        ep = jnp.exp(lp - m)
        den = (jnp.sum(ec, axis=1, keepdims=True)
               + jnp.sum(ep, axis=1, keepdims=True))
        pc = (ec / den).astype(bf16)
        pp = (ep / den).astype(bf16)
        o = _dot(pc, xvs_ref[:, sl]) + _dot(pp, xvsp_ref[:, sl])
        outs.append(o)
    ocat = jnp.concatenate(outs, axis=1).astype(bf16)
    osw_ref[...] = _dot(ocat, wo_ref[...])


def _sel_body(qraw_ref, cs_ref, kall_ref, ccomp_ref, mask_ref, sink_ref,
              qnw_ref, wo_ref, h_ref, osw_ref, out_ref):
    kb = kall_ref[...].astype(bf16)
    vb = ccomp_ref[...].astype(bf16)
    ma = mask_ref[...]
    csc = cs_ref[...]
    cos, sin = csc[:, :64], csc[:, 64:]
    outs = []
    for hh in range(H):
        q = qraw_ref[:, 64 * hh:64 * hh + 64]
        q = q * jax.lax.rsqrt(jnp.mean(q * q, axis=1, keepdims=True) + 1e-6)
        q = q * qnw_ref[...]
        q = _rope(q, cos, sin).astype(bf16)
        lg = _dot_nt(q, kb) + ma
        sk = sink_ref[0:1, hh:hh + 1]
        m = jnp.maximum(jnp.max(lg, axis=1, keepdims=True), sk)
        e = jnp.exp(lg - m)
        den = jnp.sum(e, axis=1, keepdims=True) + jnp.exp(sk - m)
        p = (e / den).astype(bf16)
        outs.append(_dot(p, vb))
    att = jnp.concatenate(outs, axis=1).astype(bf16)
    out_ref[...] = _dot(att, wo_ref[...]) + h_ref[...] + osw_ref[...]


def _full(shape):
    return pl.BlockSpec(shape, lambda i: (0, 0))


def _blk(shape):
    return pl.BlockSpec(shape, lambda i: (i, 0))


def _prev(shape):
    return pl.BlockSpec(shape, lambda i: (jnp.maximum(i - 1, 0), 0))


def kernel(h, phrase_mask, phrase_token_idx, phrase_end_pos, rope_cos,
           rope_sin, W_dq, W_uq, kv_Wkv, kv_Wz, kv_Bpos, ik_Wkv, ik_Wz,
           ik_Bpos, idx_Wiuq, idx_Ww, qn_w, kn_w, W_o, sink_logits, norm_w,
           sw_Wq, sw_Wk, sw_Wv, sw_Wo):
    h2 = h[0]
    wzc = jnp.concatenate([kv_Wkv, kv_Wz, ik_Wkv, ik_Wz], axis=1).astype(bf16)
    www = jnp.pad(idx_Ww, ((0, 0), (0, 128 - NI))).astype(bf16)
    tok3 = phrase_token_idx[0].astype(i32).reshape(NPB, PB, LMAX)
    idxp = tok3.transpose(0, 2, 1).reshape(P * LMAX, 1)
    endp_i = phrase_end_pos[0].astype(i32).reshape(P, 1)
    endp_f = phrase_end_pos[0].astype(f32).reshape(1, P)
    cs = jnp.concatenate([rope_cos, rope_sin], axis=1)
    nw = norm_w.reshape(1, D)
    qnw = qn_w.reshape(1, C)
    knw = kn_w.reshape(1, C)
    sink = sink_logits.reshape(1, H)

    zc4, qi, wh, qraw, xqs, xks, xvs = pl.pallas_call(
        _proj_body,
        grid=(NTB,),
        in_specs=[
            _blk((TB, D)), _full((1, D)), _full((D, 256)), _full((D, QCD)),
            _full((QCD, 256)), _full((QCD, D)), _full((D, 128)),
            _full((D, 512)), _full((D, 512)), _full((D, 512)),
        ],
        out_specs=[
            _blk((TB, 256)), _blk((TB, 256)), _blk((TB, 128)),
            _blk((TB, D)), _blk((TB, 512)), _blk((TB, 512)), _blk((TB, 512)),
        ],
        out_shape=[
            jax.ShapeDtypeStruct((T, 256), f32),
            jax.ShapeDtypeStruct((T, 256), bf16),
            jax.ShapeDtypeStruct((T, 128), f32),
            jax.ShapeDtypeStruct((T, D), f32),
            jax.ShapeDtypeStruct((T, 512), bf16),
            jax.ShapeDtypeStruct((T, 512), bf16),
            jax.ShapeDtypeStruct((T, 512), bf16),
        ],
    )(h2, nw, wzc, W_dq.astype(bf16), idx_Wiuq.astype(bf16),
      W_uq.astype(bf16), www, sw_Wq.astype(bf16), sw_Wk.astype(bf16),
      sw_Wv.astype(bf16))

    ccomp, kidx, kall = pl.pallas_call(
        _compress_body,
        grid=(NPB,),
        in_specs=[
            _blk((PB * LMAX, 1)), _full((T, 256)), _blk((PB, 1)),
            _full((LMAX, C)), _full((LMAX, C)), _full((T, 128)),
            _full((1, C)),
        ],
        out_specs=[_blk((PB, C)), _blk((PB, C)), _blk((PB, C))],
        out_shape=[
            jax.ShapeDtypeStruct((P, C), f32),
            jax.ShapeDtypeStruct((P, C), f32),
            jax.ShapeDtypeStruct((P, C), f32),
        ],
    )(idxp, zc4, endp_i, kv_Bpos, ik_Bpos, cs.astype(bf16), knw)

    maskadd = pl.pallas_call(
        _topk_body,
        grid=(NTB,),
        in_specs=[_blk((TB, 256)), _blk((TB, 128)), _full((P, C)),
                  _full((1, P))],
        out_specs=_blk((TB, P)),
        out_shape=jax.ShapeDtypeStruct((T, P), f32),
    )(qi, wh, kidx, endp_f)

    osw = pl.pallas_call(
        _swin_body,
        grid=(NTB,),
        in_specs=[
            _blk((TB, 512)), _blk((TB, 512)), _prev((TB, 512)),
            _blk((TB, 512)), _prev((TB, 512)), _blk((TB, 128)),
            _prev((TB, 128)), _full((512, D)),
        ],
        out_specs=_blk((TB, D)),
        out_shape=jax.ShapeDtypeStruct((T, D), f32),
    )(xqs, xks, xks, xvs, xvs, cs, cs, sw_Wo.astype(bf16))

    out = pl.pallas_call(
        _sel_body,
        grid=(NTB,),
        in_specs=[
            _blk((TB, D)), _blk((TB, 128)), _full((P, C)), _full((P, C)),
            _blk((TB, P)), _full((1, H)), _full((1, C)), _full((D, D)),
            _blk((TB, D)), _blk((TB, D)),
        ],
        out_specs=_blk((TB, D)),
        out_shape=jax.ShapeDtypeStruct((T, D), f32),
    )(qraw, cs, kall, ccomp, maskadd, sink, qnw, W_o.astype(bf16), h2, osw)

    return out.reshape(1, T, D)


# speedup vs baseline: 7.3369x; 1.2093x over previous
import math

import jax
import jax.numpy as jnp
from jax.experimental import pallas as pl

T = 2048
D = 1024
H, C = 16, 64
QCD = 256
TOPK = 32
NI, DI = 4, 64
LMAX = 16
P = 512
NWIN = 128
HS = 8

TB = 256
NTB = T // TB
PB = 128
NPB = P // PB
NEG = -1e30

f32 = jnp.float32
bf16 = jnp.bfloat16
i32 = jnp.int32


def _dot(a, b):
    return jax.lax.dot_general(a, b, (((1,), (0,)), ((), ())),
                               preferred_element_type=f32)


def _rope(x, cos, sin):
    half = x.shape[-1] // 2
    rot = jnp.concatenate([-x[:, half:], x[:, :half]], axis=1)
    return x * cos + rot * sin


def _proj_body(h_ref, nw_ref, qnw_ref, cs_ref, wzc_ref, wdq_ref, wiuq_ref,
               wuq_ref, www_ref, wsq_ref, wsk_ref, wsv_ref,
               zc4_ref, qi_ref, wh_ref, qrope_ref, xqs_ref, xksT_ref,
               xvs_ref):
    hb = h_ref[...]
    ms = jnp.mean(hb * hb, axis=1, keepdims=True)
    x = hb * jax.lax.rsqrt(ms + 1e-6) * nw_ref[...]
    xb = x.astype(bf16)
    csc = cs_ref[...]
    cos, sin = csc[:, :64], csc[:, 64:]
    zc4_ref[...] = _dot(xb, wzc_ref[...])
    qlat = _dot(xb, wdq_ref[...])
    qlb = qlat.astype(bf16)
    qi_ref[...] = _dot(qlb, wiuq_ref[...]).astype(bf16)
    wh_ref[...] = _dot(xb, www_ref[...])
    qraw = _dot(qlb, wuq_ref[...])
    qh = []
    for hh in range(H):
        q = qraw[:, 64 * hh:64 * hh + 64]
        q = q * jax.lax.rsqrt(jnp.mean(q * q, axis=1, keepdims=True) + 1e-6)
        qh.append(_rope(q * qnw_ref[...], cos, sin))
    qrope_ref[...] = jnp.concatenate(qh, axis=1).astype(bf16)
    xqs = _dot(xb, wsq_ref[...])
    xks = _dot(xb, wsk_ref[...])
    qsh, ksh = [], []
    for hh in range(HS):
        sl = slice(64 * hh, 64 * hh + 64)
        qsh.append(_rope(xqs[:, sl], cos, sin) * (1.0 / math.sqrt(C)))
        ksh.append(_rope(xks[:, sl], cos, sin))
    xqs_ref[...] = jnp.concatenate(qsh, axis=1).astype(bf16)
    xksT_ref[...] = jnp.transpose(jnp.concatenate(ksh, axis=1)).astype(bf16)
    xvs_ref[...] = _dot(xb, wsv_ref[...]).astype(bf16)


def _compress_body(idx_ref, zc4_ref, endp_ref, bkv_ref, bik_ref, cs_ref,
                   knw_ref, ccomp_ref, kidxT_ref, kallT_ref):
    idxv = idx_ref[...]
    ioh = jax.lax.broadcasted_iota(i32, (PB * LMAX, T), 1)
    oh = (idxv == ioh).astype(bf16)
    g = _dot(oh, zc4_ref[...].astype(bf16))

    def slot(l, lo):
        return g[l * PB:(l + 1) * PB, lo:lo + 64]

    mkv = jnp.full((PB, 64), NEG, f32)
    mik = jnp.full((PB, 64), NEG, f32)
    for l in range(LMAX):
        mkv = jnp.maximum(mkv, slot(l, 64) + bkv_ref[l:l + 1, :])
        mik = jnp.maximum(mik, slot(l, 192) + bik_ref[l:l + 1, :])
    skv = jnp.zeros((PB, 64), f32)
    sik = jnp.zeros((PB, 64), f32)
    akv = jnp.zeros((PB, 64), f32)
    aik = jnp.zeros((PB, 64), f32)
    for l in range(LMAX):
        ekv = jnp.exp(slot(l, 64) + bkv_ref[l:l + 1, :] - mkv)
        eik = jnp.exp(slot(l, 192) + bik_ref[l:l + 1, :] - mik)
        skv += ekv
        sik += eik
        akv += ekv * slot(l, 0)
        aik += eik * slot(l, 128)
    ccomp = akv / skv
    ccomp_ref[...] = ccomp
    kidxT_ref[...] = jnp.transpose(aik / sik)

    ohe = (endp_ref[...] == jax.lax.broadcasted_iota(i32, (PB, T), 1)).astype(bf16)
    cse = _dot(ohe, cs_ref[...])
    mean = jnp.mean(ccomp * ccomp, axis=1, keepdims=True)
    kn = ccomp * jax.lax.rsqrt(mean + 1e-6) * knw_ref[...]
    kall = _rope(kn, cse[:, :64], cse[:, 64:]) * (1.0 / math.sqrt(C))
    kallT_ref[...] = jnp.transpose(kall)


def _swin_body(xqs_ref, xksT_ref, xksTp_ref, xvs_ref, xvsp_ref, wo_ref,
               osw_ref):
    blk = pl.program_id(0)
    isub = jax.lax.broadcasted_iota(i32, (TB, TB), 0)
    jlan = jax.lax.broadcasted_iota(i32, (TB, TB), 1)
    mask_cur = (jlan <= isub) & (isub - jlan < NWIN)
    mask_prev = (isub < jlan - (TB - NWIN)) & (blk > 0)
    outs = []
    for hh in range(HS):
        sl = slice(64 * hh, 64 * hh + 64)
        q = xqs_ref[:, sl]
        lc = _dot(q, xksT_ref[sl, :])
        lp = _dot(q, xksTp_ref[sl, :])
        lc = jnp.where(mask_cur, lc, NEG)
        lp = jnp.where(mask_prev, lp, NEG)
        m = jnp.maximum(jnp.max(lc, axis=1, keepdims=True),
                        jnp.max(lp, axis=1, keepdims=True))
        ec = jnp.exp(lc - m)
        ep = jnp.exp(lp - m)
        inv = 1.0 / (jnp.sum(ec, axis=1, keepdims=True)
                     + jnp.sum(ep, axis=1, keepdims=True))
        pc = (ec * inv).astype(bf16)
        pp = (ep * inv).astype(bf16)
        outs.append(_dot(pc, xvs_ref[:, sl]) + _dot(pp, xvsp_ref[:, sl]))
    ocat = jnp.concatenate(outs, axis=1).astype(bf16)
    osw_ref[...] = _dot(ocat, wo_ref[...])


def _sel_body(qi_ref, wh_ref, kidxT_ref, endp_ref, qrope_ref, kallT_ref,
              ccomp_ref, sink_ref, wo_ref, h_ref, osw_ref, out_ref):
    kiT = kidxT_ref[...].astype(bf16)
    scores = jnp.zeros((TB, P), f32)
    for ih in range(NI):
        s = _dot(qi_ref[:, 64 * ih:64 * ih + 64], kiT)
        scores += jnp.maximum(s, 0.0) * wh_ref[:, ih:ih + 1]
    t0 = pl.program_id(0) * TB
    rowpos = (t0 + jax.lax.broadcasted_iota(i32, (TB, 1), 0)).astype(f32)
    vis = endp_ref[...] < rowpos
    scores = jnp.where(vis, scores, NEG)
    lane = jax.lax.broadcasted_iota(i32, (TB, P), 1)
    selb = jnp.zeros((TB, P), jnp.bool_)
    for _ in range(TOPK):
        m = jnp.max(scores, axis=1, keepdims=True)
        valid = m > -1e29
        ism = scores == m
        ii = jnp.min(jnp.where(ism, lane, P * 8), axis=1, keepdims=True)
        s1 = (lane == ii) & valid
        selb = selb | s1
        scores = jnp.where(s1, -3e30, scores)
    ma = jnp.where(selb, 0.0, NEG)

    kT = kallT_ref[...].astype(bf16)
    vb = ccomp_ref[...].astype(bf16)
    outs = []
    for hh in range(H):
        q = qrope_ref[:, 64 * hh:64 * hh + 64]
        lg = _dot(q, kT) + ma
        sk = sink_ref[0:1, hh:hh + 1]
        m = jnp.maximum(jnp.max(lg, axis=1, keepdims=True), sk)
        e = jnp.exp(lg - m)
        inv = 1.0 / (jnp.sum(e, axis=1, keepdims=True) + jnp.exp(sk - m))
        outs.append(_dot((e * inv).astype(bf16), vb))
    att = jnp.concatenate(outs, axis=1).astype(bf16)
    out_ref[...] = _dot(att, wo_ref[...]) + h_ref[...] + osw_ref[...]


def _full(shape):
    return pl.BlockSpec(shape, lambda i: (0, 0))


def _blk(shape):
    return pl.BlockSpec(shape, lambda i: (i, 0))


def _blkT(shape):
    return pl.BlockSpec(shape, lambda i: (0, i))


def _prevT(shape):
    return pl.BlockSpec(shape, lambda i: (0, jnp.maximum(i - 1, 0)))


def _prev(shape):
    return pl.BlockSpec(shape, lambda i: (jnp.maximum(i - 1, 0), 0))


def kernel(h, phrase_mask, phrase_token_idx, phrase_end_pos, rope_cos,
           rope_sin, W_dq, W_uq, kv_Wkv, kv_Wz, kv_Bpos, ik_Wkv, ik_Wz,
           ik_Bpos, idx_Wiuq, idx_Ww, qn_w, kn_w, W_o, sink_logits, norm_w,
           sw_Wq, sw_Wk, sw_Wv, sw_Wo):
    h2 = h[0]
    wzc = jnp.concatenate([kv_Wkv, kv_Wz, ik_Wkv, ik_Wz], axis=1).astype(bf16)
    www = jnp.pad(idx_Ww, ((0, 0), (0, 128 - NI))).astype(bf16)
    tok3 = phrase_token_idx[0].astype(i32).reshape(NPB, PB, LMAX)
    idxp = tok3.transpose(0, 2, 1).reshape(P * LMAX, 1)
    endp_i = phrase_end_pos[0].astype(i32).reshape(P, 1)
    endp_f = phrase_end_pos[0].astype(f32).reshape(1, P)
    cs = jnp.concatenate([rope_cos, rope_sin], axis=1)
    nw = norm_w.reshape(1, D)
    qnw = qn_w.reshape(1, C)
    knw = kn_w.reshape(1, C)
    sink = sink_logits.reshape(1, H)

    zc4, qi, wh, qrope, xqs, xksT, xvs = pl.pallas_call(
        _proj_body,
        grid=(NTB,),
        in_specs=[
            _blk((TB, D)), _full((1, D)), _full((1, C)), _blk((TB, 128)),
            _full((D, 256)), _full((D, QCD)), _full((QCD, 256)),
            _full((QCD, D)), _full((D, 128)), _full((D, 512)),
            _full((D, 512)), _full((D, 512)),
        ],
        out_specs=[
            _blk((TB, 256)), _blk((TB, 256)), _blk((TB, 128)),
            _blk((TB, D)), _blk((TB, 512)), _blkT((512, TB)),
            _blk((TB, 512)),
        ],
        out_shape=[
            jax.ShapeDtypeStruct((T, 256), f32),
            jax.ShapeDtypeStruct((T, 256), bf16),
            jax.ShapeDtypeStruct((T, 128), f32),
            jax.ShapeDtypeStruct((T, D), bf16),
            jax.ShapeDtypeStruct((T, 512), bf16),
            jax.ShapeDtypeStruct((512, T), bf16),
            jax.ShapeDtypeStruct((T, 512), bf16),
        ],
    )(h2, nw, qnw, cs, wzc, W_dq.astype(bf16), idx_Wiuq.astype(bf16),
      W_uq.astype(bf16), www, sw_Wq.astype(bf16), sw_Wk.astype(bf16),
      sw_Wv.astype(bf16))

    ccomp, kidxT, kallT = pl.pallas_call(
        _compress_body,
        grid=(NPB,),
        in_specs=[
            _blk((PB * LMAX, 1)), _full((T, 256)), _blk((PB, 1)),
            _full((LMAX, C)), _full((LMAX, C)), _full((T, 128)),
            _full((1, C)),
        ],
        out_specs=[_blk((PB, C)), _blkT((C, PB)), _blkT((C, PB))],
        out_shape=[
            jax.ShapeDtypeStruct((P, C), f32),
            jax.ShapeDtypeStruct((C, P), f32),
            jax.ShapeDtypeStruct((C, P), f32),
        ],
    )(idxp, zc4, endp_i, kv_Bpos, ik_Bpos, cs.astype(bf16), knw)

    osw = pl.pallas_call(
        _swin_body,
        grid=(NTB,),
        in_specs=[
            _blk((TB, 512)), _blkT((512, TB)), _prevT((512, TB)),
            _blk((TB, 512)), _prev((TB, 512)), _full((512, D)),
        ],
        out_specs=_blk((TB, D)),
        out_shape=jax.ShapeDtypeStruct((T, D), f32),
    )(xqs, xksT, xksT, xvs, xvs, sw_Wo.astype(bf16))

    out = pl.pallas_call(
        _sel_body,
        grid=(NTB,),
        in_specs=[
            _blk((TB, 256)), _blk((TB, 128)), _full((C, P)), _full((1, P)),
            _blk((TB, D)), _full((C, P)), _full((P, C)), _full((1, H)),
            _full((D, D)), _blk((TB, D)), _blk((TB, D)),
        ],
        out_specs=_blk((TB, D)),
        out_shape=jax.ShapeDtypeStruct((T, D), f32),
    )(qi, wh, kidxT, endp_f, qrope, kallT, ccomp, sink, W_o.astype(bf16),
      h2, osw)

    return out.reshape(1, T, D)
